# baseline (speedup 1.0000x reference)
import functools

import jax
import jax.numpy as jnp
from jax import lax
from jax.experimental import pallas as pl
from jax.experimental.pallas import tpu as pltpu
from jax.experimental.pallas import tpu_sc as plsc

DIM = 129
TABLE_SIZE = DIM * DIM * DIM
N_REFL = 4194304

NUM_CORES = 2
NUM_SUBCORES = 16
NUM_WORKERS = NUM_CORES * NUM_SUBCORES

PHASE_SIZES = (524288, 1835008, 1835008)
assert sum(PHASE_SIZES) == N_REFL

CHUNK = 2048
NBUF = 4


def _make_sc_gather(phase_base, phase_size):
    mesh = plsc.VectorSubcoreMesh(core_axis_name="c", subcore_axis_name="s")
    PER_WORKER = phase_size // NUM_WORKERS
    NCHUNKS = PER_WORKER // CHUNK
    ROUNDS = NCHUNKS // NBUF
    assert ROUNDS * NBUF * CHUNK * NUM_WORKERS == phase_size

    @functools.partial(
        pl.kernel,
        mesh=mesh,
        out_type=(),
        compiler_params=pltpu.CompilerParams(needs_layout_passes=False),
        scratch_types=(
            [pltpu.VMEM((CHUNK,), jnp.int32) for _ in range(NBUF)]
            + [pltpu.VMEM((CHUNK,), jnp.int32) for _ in range(NBUF)]
            + [pltpu.SemaphoreType.DMA for _ in range(3 * NBUF)]
        ),
    )
    def sc_gather(flat_hbm, table_hbm, out_ref, *scratch):
        out_hbm = out_ref
        idx_bufs = scratch[0:NBUF]
        val_bufs = scratch[NBUF : 2 * NBUF]
        in_sems = scratch[2 * NBUF : 3 * NBUF]
        g_sems = scratch[3 * NBUF : 4 * NBUF]
        o_sems = scratch[4 * NBUF : 5 * NBUF]

        wid = lax.axis_index("s") * NUM_CORES + lax.axis_index("c")
        base = wid * PER_WORKER
        obase = phase_base + wid * PER_WORKER

        def start_in(q, b):
            pltpu.async_copy(
                flat_hbm.at[pl.ds(base + q * CHUNK, CHUNK)], idx_bufs[b], in_sems[b]
            )

        def wait_in(b):
            pltpu.make_async_copy(
                flat_hbm.at[pl.ds(base, CHUNK)], idx_bufs[b], in_sems[b]
            ).wait()

        def fire_gather(b):
            pltpu.async_copy(table_hbm.at[idx_bufs[b]], val_bufs[b], g_sems[b])

        def wait_gather(b):
            pltpu.make_async_copy(
                flat_hbm.at[pl.ds(base, CHUNK)], val_bufs[b], g_sems[b]
            ).wait()

        def start_out(q, b):
            pltpu.async_copy(
                val_bufs[b], out_hbm.at[pl.ds(obase + q * CHUNK, CHUNK)], o_sems[b]
            )

        def wait_out(b):
            pltpu.make_async_copy(
                val_bufs[b], out_hbm.at[pl.ds(obase, CHUNK)], o_sems[b]
            ).wait()

        def step(q, b):
            wait_in(b)

            @pl.when(q >= NBUF)
            def _():
                wait_out(b)

            fire_gather(b)

            @pl.when(q >= 2)
            def _():
                wait_gather((b + 2) % NBUF)

            @pl.when(q >= 2)
            def _():
                start_out(q - 2, (b + 2) % NBUF)

            @pl.when(q + 2 < NCHUNKS)
            def _():
                start_in(q + 2, (b + 2) % NBUF)

        start_in(0, 0)
        start_in(1, 1)

        def body(r, carry):
            for i in range(NBUF):
                step(r * NBUF + i, i)
            return carry

        lax.fori_loop(0, ROUNDS, body, 0, unroll=False)

        for q in (NCHUNKS - 2, NCHUNKS - 1):
            b = q % NBUF
            wait_gather(b)
            start_out(q, b)
        for q in range(NCHUNKS - NBUF, NCHUNKS):
            wait_out(q % NBUF)

    return sc_gather


_PHASE_BASES = tuple(sum(PHASE_SIZES[:p]) for p in range(len(PHASE_SIZES)))
_sc_gather_phases = [
    _make_sc_gather(b, s) for b, s in zip(_PHASE_BASES, PHASE_SIZES)
]


def kernel(hkl, miller_id):
    table = miller_id.reshape(TABLE_SIZE)
    out_ref = jax.new_ref(pl.empty((N_REFL,), jnp.int32))
    for p, (b, s) in enumerate(zip(_PHASE_BASES, PHASE_SIZES)):
        src = hkl if p == 0 else lax.optimization_barrier(hkl)
        hp = lax.slice(src, (b, 0), (b + s, 3))
        h2 = hp[:, 0].reshape(-1, 128)
        k2 = hp[:, 1].reshape(-1, 128)
        l2 = hp[:, 2].reshape(-1, 128)
        flat = (h2 * (DIM * DIM) + k2 * DIM + l2).reshape(s)
        _sc_gather_phases[p](flat, table, out_ref)
    return out_ref[...]


# speedup vs baseline: 1.0616x; 1.0616x over previous
import functools

import jax
import jax.numpy as jnp
from jax import lax
from jax.experimental import pallas as pl
from jax.experimental.pallas import tpu as pltpu
from jax.experimental.pallas import tpu_sc as plsc

DIM = 129
TABLE_SIZE = DIM * DIM * DIM
N_REFL = 4194304

NUM_CORES = 2
NUM_SUBCORES = 16
NUM_WORKERS = NUM_CORES * NUM_SUBCORES

PHASE_SIZES = (1572864, 2621440)
assert sum(PHASE_SIZES) == N_REFL

CHUNK = 2048
NBUF = 4


def _make_sc_gather(phase_base, phase_size):
    mesh = plsc.VectorSubcoreMesh(core_axis_name="c", subcore_axis_name="s")
    PER_WORKER = phase_size // NUM_WORKERS
    NCHUNKS = PER_WORKER // CHUNK
    ROUNDS = NCHUNKS // NBUF
    assert ROUNDS * NBUF * CHUNK * NUM_WORKERS == phase_size

    @functools.partial(
        pl.kernel,
        mesh=mesh,
        out_type=(),
        compiler_params=pltpu.CompilerParams(needs_layout_passes=False),
        scratch_types=(
            [pltpu.VMEM((CHUNK,), jnp.int32) for _ in range(NBUF)]
            + [pltpu.VMEM((CHUNK,), jnp.int32) for _ in range(NBUF)]
            + [pltpu.SemaphoreType.DMA for _ in range(3 * NBUF)]
        ),
    )
    def sc_gather(flat_hbm, table_hbm, out_ref, *scratch):
        out_hbm = out_ref
        idx_bufs = scratch[0:NBUF]
        val_bufs = scratch[NBUF : 2 * NBUF]
        in_sems = scratch[2 * NBUF : 3 * NBUF]
        g_sems = scratch[3 * NBUF : 4 * NBUF]
        o_sems = scratch[4 * NBUF : 5 * NBUF]

        wid = lax.axis_index("s") * NUM_CORES + lax.axis_index("c")
        base = wid * PER_WORKER
        obase = phase_base + wid * PER_WORKER

        def start_in(q, b):
            pltpu.async_copy(
                flat_hbm.at[pl.ds(base + q * CHUNK, CHUNK)], idx_bufs[b], in_sems[b]
            )

        def wait_in(b):
            pltpu.make_async_copy(
                flat_hbm.at[pl.ds(base, CHUNK)], idx_bufs[b], in_sems[b]
            ).wait()

        def fire_gather(b):
            pltpu.async_copy(table_hbm.at[idx_bufs[b]], val_bufs[b], g_sems[b])

        def wait_gather(b):
            pltpu.make_async_copy(
                flat_hbm.at[pl.ds(base, CHUNK)], val_bufs[b], g_sems[b]
            ).wait()

        def start_out(q, b):
            pltpu.async_copy(
                val_bufs[b], out_hbm.at[pl.ds(obase + q * CHUNK, CHUNK)], o_sems[b]
            )

        def wait_out(b):
            pltpu.make_async_copy(
                val_bufs[b], out_hbm.at[pl.ds(obase, CHUNK)], o_sems[b]
            ).wait()

        def step(q, b):
            wait_in(b)

            @pl.when(q >= NBUF)
            def _():
                wait_out(b)

            fire_gather(b)

            @pl.when(q >= 2)
            def _():
                wait_gather((b + 2) % NBUF)

            @pl.when(q >= 2)
            def _():
                start_out(q - 2, (b + 2) % NBUF)

            @pl.when(q + 2 < NCHUNKS)
            def _():
                start_in(q + 2, (b + 2) % NBUF)

        start_in(0, 0)
        start_in(1, 1)

        def body(r, carry):
            for i in range(NBUF):
                step(r * NBUF + i, i)
            return carry

        lax.fori_loop(0, ROUNDS, body, 0, unroll=False)

        for q in (NCHUNKS - 2, NCHUNKS - 1):
            b = q % NBUF
            wait_gather(b)
            start_out(q, b)
        for q in range(NCHUNKS - NBUF, NCHUNKS):
            wait_out(q % NBUF)

    return sc_gather


_PHASE_BASES = tuple(sum(PHASE_SIZES[:p]) for p in range(len(PHASE_SIZES)))
_sc_gather_phases = [
    _make_sc_gather(b, s) for b, s in zip(_PHASE_BASES, PHASE_SIZES)
]


def kernel(hkl, miller_id):
    table = miller_id.reshape(TABLE_SIZE)
    out_ref = jax.new_ref(pl.empty((N_REFL,), jnp.int32))
    for p, (b, s) in enumerate(zip(_PHASE_BASES, PHASE_SIZES)):
        src = hkl if p == 0 else lax.optimization_barrier(hkl)
        hp = lax.slice(src, (b, 0), (b + s, 3))
        h2 = hp[:, 0].reshape(-1, 128)
        k2 = hp[:, 1].reshape(-1, 128)
        l2 = hp[:, 2].reshape(-1, 128)
        flat = (h2 * (DIM * DIM) + k2 * DIM + l2).reshape(s)
        _sc_gather_phases[p](flat, table, out_ref)
    return out_ref[...]
